# baseline (speedup 1.0000x reference)
import functools

import jax
import jax.numpy as jnp
from jax import lax
from jax.experimental import pallas as pl
from jax.experimental.pallas import tpu as pltpu
from jax.experimental.pallas import tpu_sc as plsc

N = 10000
S = 512
D = 128
E = 320000

GSZ = 32
GROUPS = 318
EPT = GROUPS * GSZ
E_PAD = 32 * EPT
DEG_PAD = 10240
N_PAD = 10240
ROWS_PT = N_PAD // 16
RBLK = 32

_vmesh = plsc.VectorSubcoreMesh(core_axis_name="c", subcore_axis_name="s")

_sc_params = pltpu.CompilerParams(
    needs_layout_passes=False, use_tc_tiling_on_sc=False)


def _deg_call(dstp, wp):
    @functools.partial(
        pl.kernel,
        out_type=jax.ShapeDtypeStruct((2, DEG_PAD), jnp.float32),
        mesh=_vmesh,
        compiler_params=_sc_params,
        scratch_types=[
            pltpu.VMEM((GROUPS, GSZ), jnp.int32),
            pltpu.VMEM((GROUPS, GSZ), jnp.float32),
            pltpu.VMEM((DEG_PAD,), jnp.float32),
            pltpu.VMEM((640,), jnp.float32),
            pltpu.VMEM_SHARED((16, DEG_PAD), jnp.float32),
        ],
    )
    def k(dst_hbm, w_hbm, out_hbm, dst_v, w_v, deg_v, tmp_v, sh):
        c = lax.axis_index("c")
        s = lax.axis_index("s")
        wid = c * 16 + s
        pltpu.sync_copy(dst_hbm.at[wid], dst_v)
        pltpu.sync_copy(w_hbm.at[wid], w_v)

        @pl.loop(0, DEG_PAD // 16)
        def _(i):
            deg_v[pl.ds(i * 16, 16)] = jnp.zeros((16,), jnp.float32)

        @pl.loop(0, GROUPS)
        def _(g):
            for kk in range(GSZ // 16):
                d16 = dst_v[g, pl.ds(kk * 16, 16)]
                w16 = w_v[g, pl.ds(kk * 16, 16)]
                plsc.addupdate_scatter(deg_v, [d16], w16)

        pltpu.sync_copy(deg_v, sh.at[s])
        plsc.subcore_barrier()

        @pl.loop(0, 40)
        def _(i):
            deg_v[pl.ds(i * 16, 16)] = jnp.zeros((16,), jnp.float32)

        for t in range(16):
            pltpu.sync_copy(sh.at[t, pl.ds(s * 640, 640)], tmp_v)

            @pl.loop(0, 40)
            def _(i):
                deg_v[pl.ds(i * 16, 16)] = (
                    deg_v[pl.ds(i * 16, 16)] + tmp_v[pl.ds(i * 16, 16)]
                )

        pltpu.sync_copy(deg_v.at[pl.ds(0, 640)], out_hbm.at[c, pl.ds(s * 640, 640)])

    return k(dstp, wp)


def _agg_call(y, srcp, dstp, wp):
    @functools.partial(
        pl.kernel,
        out_type=jax.ShapeDtypeStruct((2, N_PAD, D), jnp.float32),
        mesh=_vmesh,
        compiler_params=_sc_params,
        scratch_types=[
            pltpu.VMEM((GROUPS, GSZ), jnp.int32),
            pltpu.VMEM((GROUPS, GSZ), jnp.int32),
            pltpu.VMEM((GROUPS, GSZ), jnp.float32),
            pltpu.VMEM((GSZ, D), jnp.float32),
            pltpu.VMEM((GSZ, D), jnp.float32),
            pltpu.VMEM((GSZ, D), jnp.float32),
            pltpu.VMEM_SHARED((N_PAD, D), jnp.float32),
            pltpu.SemaphoreType.DMA,
            pltpu.SemaphoreType.DMA,
            pltpu.SemaphoreType.DMA,
            pltpu.SemaphoreType.DMA,
            pltpu.SemaphoreType.DMA,
            pltpu.SemaphoreType.DMA,
        ],
    )
    def k(y_hbm, src_hbm, dst_hbm, w_hbm, out_hbm,
          src_v, dst_v, w_v, rows0, rows1, rows2, acc_sh,
          gsem0, gsem1, gsem2, ssem0, ssem1, ssem2):
        c = lax.axis_index("c")
        s = lax.axis_index("s")
        wid = c * 16 + s

        @pl.loop(0, RBLK)
        def _(r):
            for kk in range(D // 16):
                rows0[r, pl.ds(kk * 16, 16)] = jnp.zeros((16,), jnp.float32)

        for blk in range(ROWS_PT // RBLK):
            pltpu.sync_copy(rows0, acc_sh.at[pl.ds(s * ROWS_PT + blk * RBLK, RBLK)])
        plsc.subcore_barrier()

        pltpu.sync_copy(src_hbm.at[wid], src_v)
        pltpu.sync_copy(dst_hbm.at[wid], dst_v)
        pltpu.sync_copy(w_hbm.at[wid], w_v)

        def scale(rows_v, g):
            @pl.loop(0, GSZ // 16)
            def _(e16):
                w16 = w_v[g, pl.ds(e16 * 16, 16)]
                for i in range(16):
                    ws = w16[i]
                    e = e16 * 16 + i
                    for kk in range(D // 16):
                        rows_v[e, pl.ds(kk * 16, 16)] = (
                            rows_v[e, pl.ds(kk * 16, 16)] * ws)

        rows = (rows0, rows1, rows2)
        gsems = (gsem0, gsem1, gsem2)
        ssems = (ssem0, ssem1, ssem2)

        for j in range(3):
            pltpu.async_copy(y_hbm.at[src_v.at[j]], rows[j], gsems[j])

        @pl.loop(0, GROUPS, step=3)
        def _(g):
            for j in range(3):
                gg = g + j
                jn = (j + 1) % 3
                pltpu.make_async_copy(y_hbm.at[src_v.at[gg]], rows[j],
                                      gsems[j]).wait()
                scale(rows[j], gg)
                pltpu.async_copy(rows[j], acc_sh.at[dst_v.at[gg]], ssems[j],
                                 add=True)

                @pl.when(gg >= 2)
                def _():
                    pltpu.make_async_copy(rows[jn], acc_sh.at[dst_v.at[0]],
                                          ssems[jn]).wait()

                @pl.when(jnp.logical_and(gg >= 2, gg + 1 < GROUPS))
                def _():
                    pltpu.async_copy(y_hbm.at[src_v.at[gg + 1]], rows[jn],
                                     gsems[jn])

        pltpu.make_async_copy(rows1, acc_sh.at[dst_v.at[0]], ssem1).wait()
        pltpu.make_async_copy(rows2, acc_sh.at[dst_v.at[0]], ssem2).wait()

        plsc.subcore_barrier()
        for blk in range(ROWS_PT // RBLK):
            base = s * ROWS_PT + blk * RBLK
            pltpu.sync_copy(acc_sh.at[pl.ds(base, RBLK)], rows0)
            pltpu.sync_copy(rows0, out_hbm.at[c, pl.ds(base, RBLK)])

    return k(y, srcp, dstp, wp)


_BLK = 1000


def _proj_body(init_ref, embs_ref, out_ref):
    init = init_ref[...]
    mask = (init != 0.0).astype(jnp.float32)
    cnt = jnp.sum(mask, axis=1, keepdims=True)
    acc = jnp.dot(mask, embs_ref[...], preferred_element_type=jnp.float32)
    out = acc / jnp.maximum(cnt, 1.0)
    out_ref[...] = jnp.where(cnt > 0, out, 0.0)


def _projection(init, params):
    idxs = jnp.arange(S, dtype=jnp.float32)[:, None]
    h = jax.nn.relu(idxs @ params['proj_W1'].T + params['proj_b1'])
    embs = h @ params['proj_W2'].T + params['proj_b2']
    return pl.pallas_call(
        _proj_body,
        grid=(N // _BLK,),
        in_specs=[
            pl.BlockSpec((_BLK, S), lambda i: (i, 0)),
            pl.BlockSpec((S, D), lambda i: (0, 0)),
        ],
        out_specs=pl.BlockSpec((_BLK, D), lambda i: (i, 0)),
        out_shape=jax.ShapeDtypeStruct((N, D), jnp.float32),
    )(init, embs)


def _scale_matmul_body(x_ref, dinv_ref, wt_ref, out_ref):
    out_ref[...] = dinv_ref[...] * jnp.dot(
        x_ref[...], wt_ref[...], preferred_element_type=jnp.float32)


def _scale_matmul(x, dinv, wt):
    return pl.pallas_call(
        _scale_matmul_body,
        grid=(N // _BLK,),
        in_specs=[
            pl.BlockSpec((_BLK, D), lambda i: (i, 0)),
            pl.BlockSpec((_BLK, 1), lambda i: (i, 0)),
            pl.BlockSpec((D, D), lambda i: (0, 0)),
        ],
        out_specs=pl.BlockSpec((_BLK, D), lambda i: (i, 0)),
        out_shape=jax.ShapeDtypeStruct((N, D), jnp.float32),
    )(x, dinv, wt)


def _layer_body(p_ref, y_ref, dinv_ref, b_ref, wt_ref, out_ref):
    dinv = dinv_ref[...]
    h = jax.nn.relu(dinv * (p_ref[0] + p_ref[1] + y_ref[...]) + b_ref[...])
    out_ref[...] = dinv * jnp.dot(h, wt_ref[...], preferred_element_type=jnp.float32)


def _layer(parts, y, dinv, b, wt_next):
    return pl.pallas_call(
        _layer_body,
        grid=(N // _BLK,),
        in_specs=[
            pl.BlockSpec((2, _BLK, D), lambda i: (0, i, 0)),
            pl.BlockSpec((_BLK, D), lambda i: (i, 0)),
            pl.BlockSpec((_BLK, 1), lambda i: (i, 0)),
            pl.BlockSpec((1, D), lambda i: (0, 0)),
            pl.BlockSpec((D, D), lambda i: (0, 0)),
        ],
        out_specs=pl.BlockSpec((_BLK, D), lambda i: (i, 0)),
        out_shape=jax.ShapeDtypeStruct((N, D), jnp.float32),
    )(parts, y, dinv, b, wt_next)


def _layer_last_body(p_ref, y_ref, dinv_ref, b_ref, out_ref):
    dinv = dinv_ref[...]
    out_ref[...] = jax.nn.relu(
        dinv * (p_ref[0] + p_ref[1] + y_ref[...]) + b_ref[...])


def _layer_last(parts, y, dinv, b):
    return pl.pallas_call(
        _layer_last_body,
        grid=(N // _BLK,),
        in_specs=[
            pl.BlockSpec((2, _BLK, D), lambda i: (0, i, 0)),
            pl.BlockSpec((_BLK, D), lambda i: (i, 0)),
            pl.BlockSpec((_BLK, 1), lambda i: (i, 0)),
            pl.BlockSpec((1, D), lambda i: (0, 0)),
        ],
        out_specs=pl.BlockSpec((_BLK, D), lambda i: (i, 0)),
        out_shape=jax.ShapeDtypeStruct((N, D), jnp.float32),
    )(parts, y, dinv, b)


def _att_body(h0_ref, h1_ref, h2_ref, aw_ref, ab_ref, out_ref):
    aw = aw_ref[...]
    ab = ab_ref[...]
    h0, h1, h2 = h0_ref[...], h1_ref[...], h2_ref[...]
    s0 = jnp.sum(h0 * aw, axis=1, keepdims=True) + ab
    s1 = jnp.sum(h1 * aw, axis=1, keepdims=True) + ab
    s2 = jnp.sum(h2 * aw, axis=1, keepdims=True) + ab
    m = jnp.maximum(jnp.maximum(s0, s1), s2)
    e0 = jnp.exp(s0 - m)
    e1 = jnp.exp(s1 - m)
    e2 = jnp.exp(s2 - m)
    z = e0 + e1 + e2
    out_ref[...] = (e0 * h0 + e1 * h1 + e2 * h2) / z


def _attention(hs, att_w, att_b):
    return pl.pallas_call(
        _att_body,
        grid=(N // _BLK,),
        in_specs=[
            pl.BlockSpec((_BLK, D), lambda i: (i, 0)),
            pl.BlockSpec((_BLK, D), lambda i: (i, 0)),
            pl.BlockSpec((_BLK, D), lambda i: (i, 0)),
            pl.BlockSpec((1, D), lambda i: (0, 0)),
            pl.BlockSpec((1, 1), lambda i: (0, 0)),
        ],
        out_specs=pl.BlockSpec((_BLK, D), lambda i: (i, 0)),
        out_shape=jax.ShapeDtypeStruct((N, D), jnp.float32),
    )(hs[0], hs[1], hs[2], att_w, att_b)


def _pad_edges(ei, ea):
    src = ei[0].astype(jnp.int32)
    dst = ei[1].astype(jnp.int32)
    pad = E_PAD - E
    srcp = jnp.concatenate([src, jnp.zeros((pad,), jnp.int32)]).reshape(32, GROUPS, GSZ)
    dstp = jnp.concatenate([dst, jnp.zeros((pad,), jnp.int32)]).reshape(32, GROUPS, GSZ)
    wp = jnp.concatenate([ea, jnp.zeros((pad,), jnp.float32)]).reshape(32, GROUPS, GSZ)
    return srcp, dstp, wp


def kernel(init, edge_index_cc, edge_attr_cc, edge_index_cac, edge_attr_cac,
           edge_index_csc, edge_attr_csc, params):
    p = params
    x0 = _projection(init, p)
    hs = []
    for g, ei, ea in (('cc', edge_index_cc, edge_attr_cc),
                      ('cac', edge_index_cac, edge_attr_cac),
                      ('csc', edge_index_csc, edge_attr_csc)):
        srcp, dstp, wp = _pad_edges(ei, ea)
        degp = _deg_call(dstp, wp)
        dinv = lax.rsqrt(1.0 + degp[0, :N] + degp[1, :N])[:, None]
        y = _scale_matmul(x0, dinv, p[f'{g}_W0'].T)
        for l in range(3):
            parts = _agg_call(y, srcp, dstp, wp)
            b = p[f'{g}_b{l}'][None, :]
            if l < 2:
                y = _layer(parts, y, dinv, b, p[f'{g}_W{l + 1}'].T)
            else:
                hs.append(_layer_last(parts, y, dinv, b))
    return _attention(hs, p['att_W'], p['att_b'][None, :][:, :1])


# speedup vs baseline: 1.6940x; 1.6940x over previous
import functools

import jax
import jax.numpy as jnp
from jax import lax
from jax.experimental import pallas as pl
from jax.experimental.pallas import tpu as pltpu
from jax.experimental.pallas import tpu_sc as plsc

N = 10000
S = 512
D = 128
E = 320000

GSZ = 64
GROUPS = 158
EPT = GROUPS * GSZ
E_PAD = 32 * EPT
DEG_PAD = 10240
N_PAD = 10240
ROWS_PT = N_PAD // 16
RBLK = 64

_vmesh = plsc.VectorSubcoreMesh(core_axis_name="c", subcore_axis_name="s")

_sc_params = pltpu.CompilerParams(
    needs_layout_passes=False, use_tc_tiling_on_sc=False)


def _deg_call3(dsts, ws):
    @functools.partial(
        pl.kernel,
        out_type=jax.ShapeDtypeStruct((3, 2, DEG_PAD), jnp.float32),
        mesh=_vmesh,
        compiler_params=_sc_params,
        scratch_types=[
            pltpu.VMEM((GROUPS, GSZ), jnp.int32),
            pltpu.VMEM((GROUPS, GSZ), jnp.float32),
            pltpu.VMEM((DEG_PAD,), jnp.float32),
            pltpu.VMEM((640,), jnp.float32),
            pltpu.VMEM_SHARED((16, DEG_PAD), jnp.float32),
        ],
    )
    def k(d0, d1, d2, w0, w1, w2, out_hbm, dst_v, w_v, deg_v, tmp_v, sh):
        c = lax.axis_index("c")
        s = lax.axis_index("s")
        wid = c * 16 + s
        for q, (dh, wh) in enumerate(zip((d0, d1, d2), (w0, w1, w2))):
            pltpu.sync_copy(dh.at[wid], dst_v)
            pltpu.sync_copy(wh.at[wid], w_v)

            @pl.loop(0, DEG_PAD // 16)
            def _(i):
                deg_v[pl.ds(i * 16, 16)] = jnp.zeros((16,), jnp.float32)

            @pl.loop(0, GROUPS)
            def _(g):
                for kk in range(GSZ // 16):
                    d16 = dst_v[g, pl.ds(kk * 16, 16)]
                    w16 = w_v[g, pl.ds(kk * 16, 16)]
                    plsc.addupdate_scatter(deg_v, [d16], w16)

            pltpu.sync_copy(deg_v, sh.at[s])
            plsc.subcore_barrier()

            @pl.loop(0, 40)
            def _(i):
                deg_v[pl.ds(i * 16, 16)] = jnp.zeros((16,), jnp.float32)

            for t in range(16):
                pltpu.sync_copy(sh.at[t, pl.ds(s * 640, 640)], tmp_v)

                @pl.loop(0, 40)
                def _(i):
                    deg_v[pl.ds(i * 16, 16)] = (
                        deg_v[pl.ds(i * 16, 16)] + tmp_v[pl.ds(i * 16, 16)]
                    )

            pltpu.sync_copy(deg_v.at[pl.ds(0, 640)],
                            out_hbm.at[q, c, pl.ds(s * 640, 640)])
            plsc.subcore_barrier()

    return k(dsts[0], dsts[1], dsts[2], ws[0], ws[1], ws[2])


def _agg_call3(ys, srcs, dsts, ws):
    @functools.partial(
        pl.kernel,
        out_type=jax.ShapeDtypeStruct((3, 2, N_PAD, D), jnp.float32),
        mesh=_vmesh,
        compiler_params=_sc_params,
        scratch_types=[
            pltpu.VMEM((GROUPS, GSZ), jnp.int32),
            pltpu.VMEM((GROUPS, GSZ), jnp.int32),
            pltpu.VMEM((GROUPS, GSZ), jnp.float32),
            pltpu.VMEM((GSZ, D), jnp.float32),
            pltpu.VMEM((GSZ, D), jnp.float32),
            pltpu.VMEM_SHARED((N_PAD, D), jnp.float32),
            pltpu.SemaphoreType.DMA,
            pltpu.SemaphoreType.DMA,
        ],
    )
    def k(y0, y1, y2, s0, s1, s2, d0, d1, d2, w0, w1, w2, out_hbm,
          src_v, dst_v, w_v, rows0, rows1, acc_sh, sem0, sem1):
        c = lax.axis_index("c")
        s = lax.axis_index("s")
        wid = c * 16 + s

        def scale(rows_v, g):
            @pl.loop(0, GSZ // 16)
            def _(e16):
                w16 = w_v[g, pl.ds(e16 * 16, 16)]
                for i in range(16):
                    ws = w16[i]
                    e = e16 * 16 + i
                    for kk in range(D // 16):
                        rows_v[e, pl.ds(kk * 16, 16)] = (
                            rows_v[e, pl.ds(kk * 16, 16)] * ws)

        for q, (y_hbm, src_hbm, dst_hbm, w_hbm) in enumerate(
                zip((y0, y1, y2), (s0, s1, s2), (d0, d1, d2), (w0, w1, w2))):
            pltpu.sync_copy(src_hbm.at[wid], src_v)
            pltpu.sync_copy(dst_hbm.at[wid], dst_v)
            pltpu.sync_copy(w_hbm.at[wid], w_v)

            @pl.loop(0, RBLK)
            def _(r):
                for kk in range(D // 16):
                    rows0[r, pl.ds(kk * 16, 16)] = jnp.zeros((16,), jnp.float32)

            for blk in range(10):
                pltpu.sync_copy(rows0,
                                acc_sh.at[pl.ds(s * ROWS_PT + blk * RBLK, RBLK)])
            plsc.subcore_barrier()

            pltpu.async_copy(y_hbm.at[src_v.at[0]], rows0, sem0)

            @pl.loop(0, GROUPS, step=2)
            def _(g):
                pltpu.async_copy(y_hbm.at[src_v.at[g + 1]], rows1, sem1)
                pltpu.make_async_copy(y_hbm.at[src_v.at[g]], rows0, sem0).wait()
                scale(rows0, g)
                pltpu.sync_copy(rows0, acc_sh.at[dst_v.at[g]], add=True)

                @pl.when(g + 2 < GROUPS)
                def _():
                    pltpu.async_copy(y_hbm.at[src_v.at[g + 2]], rows0, sem0)

                pltpu.make_async_copy(y_hbm.at[src_v.at[g + 1]], rows1,
                                      sem1).wait()
                scale(rows1, g + 1)
                pltpu.sync_copy(rows1, acc_sh.at[dst_v.at[g + 1]], add=True)

            plsc.subcore_barrier()
            for blk in range(10):
                base = s * ROWS_PT + blk * RBLK
                pltpu.sync_copy(acc_sh.at[pl.ds(base, RBLK)], rows0)
                pltpu.sync_copy(rows0, out_hbm.at[q, c, pl.ds(base, RBLK)])

    return k(ys[0], ys[1], ys[2], srcs[0], srcs[1], srcs[2],
             dsts[0], dsts[1], dsts[2], ws[0], ws[1], ws[2])


_BLK = 1000


def _proj_body(init_ref, embs_ref, out_ref):
    init = init_ref[...]
    mask = (init != 0.0).astype(jnp.float32)
    cnt = jnp.sum(mask, axis=1, keepdims=True)
    acc = jnp.dot(mask, embs_ref[...], preferred_element_type=jnp.float32)
    out = acc / jnp.maximum(cnt, 1.0)
    out_ref[...] = jnp.where(cnt > 0, out, 0.0)


def _projection(init, params):
    idxs = jnp.arange(S, dtype=jnp.float32)[:, None]
    h = jax.nn.relu(idxs @ params['proj_W1'].T + params['proj_b1'])
    embs = h @ params['proj_W2'].T + params['proj_b2']
    return pl.pallas_call(
        _proj_body,
        grid=(N // _BLK,),
        in_specs=[
            pl.BlockSpec((_BLK, S), lambda i: (i, 0)),
            pl.BlockSpec((S, D), lambda i: (0, 0)),
        ],
        out_specs=pl.BlockSpec((_BLK, D), lambda i: (i, 0)),
        out_shape=jax.ShapeDtypeStruct((N, D), jnp.float32),
    )(init, embs)


def _scale_matmul_body(x_ref, dinv_ref, wt_ref, out_ref):
    out_ref[...] = dinv_ref[...] * jnp.dot(
        x_ref[...], wt_ref[...], preferred_element_type=jnp.float32)


def _scale_matmul(x, dinv, wt):
    return pl.pallas_call(
        _scale_matmul_body,
        grid=(N // _BLK,),
        in_specs=[
            pl.BlockSpec((_BLK, D), lambda i: (i, 0)),
            pl.BlockSpec((_BLK, 1), lambda i: (i, 0)),
            pl.BlockSpec((D, D), lambda i: (0, 0)),
        ],
        out_specs=pl.BlockSpec((_BLK, D), lambda i: (i, 0)),
        out_shape=jax.ShapeDtypeStruct((N, D), jnp.float32),
    )(x, dinv, wt)


def _layer_body(p_ref, y_ref, dinv_ref, b_ref, wt_ref, out_ref):
    dinv = dinv_ref[...]
    h = jax.nn.relu(dinv * (p_ref[0, 0] + p_ref[0, 1] + y_ref[...]) + b_ref[...])
    out_ref[...] = dinv * jnp.dot(h, wt_ref[...], preferred_element_type=jnp.float32)


def _layer(parts, q, y, dinv, b, wt_next):
    return pl.pallas_call(
        _layer_body,
        grid=(N // _BLK,),
        in_specs=[
            pl.BlockSpec((1, 2, _BLK, D), lambda i, q=q: (q, 0, i, 0)),
            pl.BlockSpec((_BLK, D), lambda i: (i, 0)),
            pl.BlockSpec((_BLK, 1), lambda i: (i, 0)),
            pl.BlockSpec((1, D), lambda i: (0, 0)),
            pl.BlockSpec((D, D), lambda i: (0, 0)),
        ],
        out_specs=pl.BlockSpec((_BLK, D), lambda i: (i, 0)),
        out_shape=jax.ShapeDtypeStruct((N, D), jnp.float32),
    )(parts, y, dinv, b, wt_next)


def _layer_last_body(p_ref, y_ref, dinv_ref, b_ref, out_ref):
    dinv = dinv_ref[...]
    out_ref[...] = jax.nn.relu(
        dinv * (p_ref[0, 0] + p_ref[0, 1] + y_ref[...]) + b_ref[...])


def _layer_last(parts, q, y, dinv, b):
    return pl.pallas_call(
        _layer_last_body,
        grid=(N // _BLK,),
        in_specs=[
            pl.BlockSpec((1, 2, _BLK, D), lambda i, q=q: (q, 0, i, 0)),
            pl.BlockSpec((_BLK, D), lambda i: (i, 0)),
            pl.BlockSpec((_BLK, 1), lambda i: (i, 0)),
            pl.BlockSpec((1, D), lambda i: (0, 0)),
        ],
        out_specs=pl.BlockSpec((_BLK, D), lambda i: (i, 0)),
        out_shape=jax.ShapeDtypeStruct((N, D), jnp.float32),
    )(parts, y, dinv, b)


def _att_body(h0_ref, h1_ref, h2_ref, aw_ref, ab_ref, out_ref):
    aw = aw_ref[...]
    ab = ab_ref[...]
    h0, h1, h2 = h0_ref[...], h1_ref[...], h2_ref[...]
    s0 = jnp.sum(h0 * aw, axis=1, keepdims=True) + ab
    s1 = jnp.sum(h1 * aw, axis=1, keepdims=True) + ab
    s2 = jnp.sum(h2 * aw, axis=1, keepdims=True) + ab
    m = jnp.maximum(jnp.maximum(s0, s1), s2)
    e0 = jnp.exp(s0 - m)
    e1 = jnp.exp(s1 - m)
    e2 = jnp.exp(s2 - m)
    z = e0 + e1 + e2
    out_ref[...] = (e0 * h0 + e1 * h1 + e2 * h2) / z


def _attention(hs, att_w, att_b):
    return pl.pallas_call(
        _att_body,
        grid=(N // _BLK,),
        in_specs=[
            pl.BlockSpec((_BLK, D), lambda i: (i, 0)),
            pl.BlockSpec((_BLK, D), lambda i: (i, 0)),
            pl.BlockSpec((_BLK, D), lambda i: (i, 0)),
            pl.BlockSpec((1, D), lambda i: (0, 0)),
            pl.BlockSpec((1, 1), lambda i: (0, 0)),
        ],
        out_specs=pl.BlockSpec((_BLK, D), lambda i: (i, 0)),
        out_shape=jax.ShapeDtypeStruct((N, D), jnp.float32),
    )(hs[0], hs[1], hs[2], att_w, att_b)


def _pad_edges(ei, ea):
    src = ei[0].astype(jnp.int32)
    dst = ei[1].astype(jnp.int32)
    pad = E_PAD - E
    srcp = jnp.concatenate([src, jnp.zeros((pad,), jnp.int32)]).reshape(32, GROUPS, GSZ)
    dstp = jnp.concatenate([dst, jnp.zeros((pad,), jnp.int32)]).reshape(32, GROUPS, GSZ)
    wp = jnp.concatenate([ea, jnp.zeros((pad,), jnp.float32)]).reshape(32, GROUPS, GSZ)
    return srcp, dstp, wp


def kernel(init, edge_index_cc, edge_attr_cc, edge_index_cac, edge_attr_cac,
           edge_index_csc, edge_attr_csc, params):
    p = params
    x0 = _projection(init, p)
    names = ('cc', 'cac', 'csc')
    srcs, dsts, ws = [], [], []
    for ei, ea in ((edge_index_cc, edge_attr_cc),
                   (edge_index_cac, edge_attr_cac),
                   (edge_index_csc, edge_attr_csc)):
        srcp, dstp, wp = _pad_edges(ei, ea)
        srcs.append(srcp)
        dsts.append(dstp)
        ws.append(wp)
    deg3 = _deg_call3(dsts, ws)
    dinvs = [lax.rsqrt(1.0 + deg3[q, 0, :N] + deg3[q, 1, :N])[:, None]
             for q in range(3)]
    ys = [_scale_matmul(x0, dinvs[q], p[f'{names[q]}_W0'].T) for q in range(3)]
    hs = [None, None, None]
    for l in range(3):
        parts3 = _agg_call3(ys, srcs, dsts, ws)
        for q in range(3):
            b = p[f'{names[q]}_b{l}'][None, :]
            if l < 2:
                ys[q] = _layer(parts3, q, ys[q], dinvs[q], b,
                               p[f'{names[q]}_W{l + 1}'].T)
            else:
                hs[q] = _layer_last(parts3, q, ys[q], dinvs[q], b)
    return _attention(hs, p['att_W'], p['att_b'][None, :][:, :1])


# speedup vs baseline: 1.8007x; 1.0630x over previous
import functools

import jax
import jax.numpy as jnp
from jax import lax
from jax.experimental import pallas as pl
from jax.experimental.pallas import tpu as pltpu
from jax.experimental.pallas import tpu_sc as plsc

N = 10000
S = 512
D = 128
E = 320000

GSZ = 64
GROUPS = 158
EPT = GROUPS * GSZ
E_PAD = 32 * EPT
DEG_PAD = 10240
N_PAD = 10240
ROWS_PT = N_PAD // 16
RBLK = 64

_vmesh = plsc.VectorSubcoreMesh(core_axis_name="c", subcore_axis_name="s")

_sc_params = pltpu.CompilerParams(
    needs_layout_passes=False, use_tc_tiling_on_sc=False)


def _deg_call(dstp, wp):
    @functools.partial(
        pl.kernel,
        out_type=jax.ShapeDtypeStruct((2, DEG_PAD), jnp.float32),
        mesh=_vmesh,
        compiler_params=_sc_params,
        scratch_types=[
            pltpu.VMEM((GROUPS, GSZ), jnp.int32),
            pltpu.VMEM((GROUPS, GSZ), jnp.float32),
            pltpu.VMEM((DEG_PAD,), jnp.float32),
            pltpu.VMEM((640,), jnp.float32),
            pltpu.VMEM_SHARED((16, DEG_PAD), jnp.float32),
        ],
    )
    def k(dst_hbm, w_hbm, out_hbm, dst_v, w_v, deg_v, tmp_v, sh):
        c = lax.axis_index("c")
        s = lax.axis_index("s")
        wid = c * 16 + s
        pltpu.sync_copy(dst_hbm.at[wid], dst_v)
        pltpu.sync_copy(w_hbm.at[wid], w_v)

        @pl.loop(0, DEG_PAD // 16)
        def _(i):
            deg_v[pl.ds(i * 16, 16)] = jnp.zeros((16,), jnp.float32)

        @pl.loop(0, GROUPS)
        def _(g):
            for kk in range(GSZ // 16):
                d16 = dst_v[g, pl.ds(kk * 16, 16)]
                w16 = w_v[g, pl.ds(kk * 16, 16)]
                plsc.addupdate_scatter(deg_v, [d16], w16)

        pltpu.sync_copy(deg_v, sh.at[s])
        plsc.subcore_barrier()

        @pl.loop(0, 40)
        def _(i):
            deg_v[pl.ds(i * 16, 16)] = jnp.zeros((16,), jnp.float32)

        for t in range(16):
            pltpu.sync_copy(sh.at[t, pl.ds(s * 640, 640)], tmp_v)

            @pl.loop(0, 40)
            def _(i):
                deg_v[pl.ds(i * 16, 16)] = (
                    deg_v[pl.ds(i * 16, 16)] + tmp_v[pl.ds(i * 16, 16)]
                )

        pltpu.sync_copy(deg_v.at[pl.ds(0, 640)], out_hbm.at[c, pl.ds(s * 640, 640)])

    return k(dstp, wp)


def _agg_call(y, srcp, dstp, wp):
    @functools.partial(
        pl.kernel,
        out_type=jax.ShapeDtypeStruct((2, N_PAD, D), jnp.float32),
        mesh=_vmesh,
        compiler_params=_sc_params,
        scratch_types=[
            pltpu.VMEM((GROUPS, GSZ), jnp.int32),
            pltpu.VMEM((GROUPS, GSZ), jnp.int32),
            pltpu.VMEM((GROUPS, GSZ), jnp.float32),
            pltpu.VMEM((GSZ, D), jnp.float32),
            pltpu.VMEM((GSZ, D), jnp.float32),
            pltpu.VMEM_SHARED((N_PAD, D), jnp.float32),
            pltpu.SemaphoreType.DMA,
            pltpu.SemaphoreType.DMA,
        ],
    )
    def k(y_hbm, src_hbm, dst_hbm, w_hbm, out_hbm,
          src_v, dst_v, w_v, rows0, rows1, acc_sh, sem0, sem1):
        c = lax.axis_index("c")
        s = lax.axis_index("s")
        wid = c * 16 + s

        @pl.loop(0, RBLK)
        def _(r):
            for kk in range(D // 16):
                rows0[r, pl.ds(kk * 16, 16)] = jnp.zeros((16,), jnp.float32)

        for blk in range(10):
            pltpu.sync_copy(rows0, acc_sh.at[pl.ds(s * ROWS_PT + blk * RBLK, RBLK)])
        plsc.subcore_barrier()

        pltpu.sync_copy(src_hbm.at[wid], src_v)
        pltpu.sync_copy(dst_hbm.at[wid], dst_v)
        pltpu.sync_copy(w_hbm.at[wid], w_v)

        def scale(rows_v, g):
            @pl.loop(0, GSZ // 16)
            def _(e16):
                w16 = w_v[g, pl.ds(e16 * 16, 16)]
                for i in range(16):
                    ws = w16[i]
                    e = e16 * 16 + i
                    for kk in range(D // 16):
                        rows_v[e, pl.ds(kk * 16, 16)] = (
                            rows_v[e, pl.ds(kk * 16, 16)] * ws)

        pltpu.async_copy(y_hbm.at[src_v.at[0]], rows0, sem0)

        @pl.loop(0, GROUPS, step=2)
        def _(g):
            pltpu.async_copy(y_hbm.at[src_v.at[g + 1]], rows1, sem1)
            pltpu.make_async_copy(y_hbm.at[src_v.at[g]], rows0, sem0).wait()
            scale(rows0, g)
            pltpu.sync_copy(rows0, acc_sh.at[dst_v.at[g]], add=True)

            @pl.when(g + 2 < GROUPS)
            def _():
                pltpu.async_copy(y_hbm.at[src_v.at[g + 2]], rows0, sem0)

            pltpu.make_async_copy(y_hbm.at[src_v.at[g + 1]], rows1, sem1).wait()
            scale(rows1, g + 1)
            pltpu.sync_copy(rows1, acc_sh.at[dst_v.at[g + 1]], add=True)

        plsc.subcore_barrier()
        for blk in range(10):
            base = s * ROWS_PT + blk * RBLK
            pltpu.sync_copy(acc_sh.at[pl.ds(base, RBLK)], rows0)
            pltpu.sync_copy(rows0, out_hbm.at[c, pl.ds(base, RBLK)])

    return k(y, srcp, dstp, wp)


_BLK = 1000


def _proj_body(init_ref, embs_ref, out_ref):
    init = init_ref[...]
    mask = (init != 0.0).astype(jnp.float32)
    cnt = jnp.sum(mask, axis=1, keepdims=True)
    acc = jnp.dot(mask, embs_ref[...], preferred_element_type=jnp.float32)
    out = acc / jnp.maximum(cnt, 1.0)
    out_ref[...] = jnp.where(cnt > 0, out, 0.0)


def _projection(init, params):
    idxs = jnp.arange(S, dtype=jnp.float32)[:, None]
    h = jax.nn.relu(idxs @ params['proj_W1'].T + params['proj_b1'])
    embs = h @ params['proj_W2'].T + params['proj_b2']
    return pl.pallas_call(
        _proj_body,
        grid=(N // _BLK,),
        in_specs=[
            pl.BlockSpec((_BLK, S), lambda i: (i, 0)),
            pl.BlockSpec((S, D), lambda i: (0, 0)),
        ],
        out_specs=pl.BlockSpec((_BLK, D), lambda i: (i, 0)),
        out_shape=jax.ShapeDtypeStruct((N, D), jnp.float32),
    )(init, embs)


def _scale_matmul_body(x_ref, dinv_ref, wt_ref, out_ref):
    out_ref[...] = dinv_ref[...] * jnp.dot(
        x_ref[...], wt_ref[...], preferred_element_type=jnp.float32)


def _scale_matmul(x, dinv, wt):
    return pl.pallas_call(
        _scale_matmul_body,
        grid=(N // _BLK,),
        in_specs=[
            pl.BlockSpec((_BLK, D), lambda i: (i, 0)),
            pl.BlockSpec((_BLK, 1), lambda i: (i, 0)),
            pl.BlockSpec((D, D), lambda i: (0, 0)),
        ],
        out_specs=pl.BlockSpec((_BLK, D), lambda i: (i, 0)),
        out_shape=jax.ShapeDtypeStruct((N, D), jnp.float32),
    )(x, dinv, wt)


def _layer_body(p_ref, y_ref, dinv_ref, b_ref, wt_ref, out_ref):
    dinv = dinv_ref[...]
    h = jax.nn.relu(dinv * (p_ref[0] + p_ref[1] + y_ref[...]) + b_ref[...])
    out_ref[...] = dinv * jnp.dot(h, wt_ref[...], preferred_element_type=jnp.float32)


def _layer(parts, y, dinv, b, wt_next):
    return pl.pallas_call(
        _layer_body,
        grid=(N // _BLK,),
        in_specs=[
            pl.BlockSpec((2, _BLK, D), lambda i: (0, i, 0)),
            pl.BlockSpec((_BLK, D), lambda i: (i, 0)),
            pl.BlockSpec((_BLK, 1), lambda i: (i, 0)),
            pl.BlockSpec((1, D), lambda i: (0, 0)),
            pl.BlockSpec((D, D), lambda i: (0, 0)),
        ],
        out_specs=pl.BlockSpec((_BLK, D), lambda i: (i, 0)),
        out_shape=jax.ShapeDtypeStruct((N, D), jnp.float32),
    )(parts, y, dinv, b, wt_next)


def _layer_last_body(p_ref, y_ref, dinv_ref, b_ref, out_ref):
    dinv = dinv_ref[...]
    out_ref[...] = jax.nn.relu(
        dinv * (p_ref[0] + p_ref[1] + y_ref[...]) + b_ref[...])


def _layer_last(parts, y, dinv, b):
    return pl.pallas_call(
        _layer_last_body,
        grid=(N // _BLK,),
        in_specs=[
            pl.BlockSpec((2, _BLK, D), lambda i: (0, i, 0)),
            pl.BlockSpec((_BLK, D), lambda i: (i, 0)),
            pl.BlockSpec((_BLK, 1), lambda i: (i, 0)),
            pl.BlockSpec((1, D), lambda i: (0, 0)),
        ],
        out_specs=pl.BlockSpec((_BLK, D), lambda i: (i, 0)),
        out_shape=jax.ShapeDtypeStruct((N, D), jnp.float32),
    )(parts, y, dinv, b)


def _att_body(h0_ref, h1_ref, h2_ref, aw_ref, ab_ref, out_ref):
    aw = aw_ref[...]
    ab = ab_ref[...]
    h0, h1, h2 = h0_ref[...], h1_ref[...], h2_ref[...]
    s0 = jnp.sum(h0 * aw, axis=1, keepdims=True) + ab
    s1 = jnp.sum(h1 * aw, axis=1, keepdims=True) + ab
    s2 = jnp.sum(h2 * aw, axis=1, keepdims=True) + ab
    m = jnp.maximum(jnp.maximum(s0, s1), s2)
    e0 = jnp.exp(s0 - m)
    e1 = jnp.exp(s1 - m)
    e2 = jnp.exp(s2 - m)
    z = e0 + e1 + e2
    out_ref[...] = (e0 * h0 + e1 * h1 + e2 * h2) / z


def _attention(hs, att_w, att_b):
    return pl.pallas_call(
        _att_body,
        grid=(N // _BLK,),
        in_specs=[
            pl.BlockSpec((_BLK, D), lambda i: (i, 0)),
            pl.BlockSpec((_BLK, D), lambda i: (i, 0)),
            pl.BlockSpec((_BLK, D), lambda i: (i, 0)),
            pl.BlockSpec((1, D), lambda i: (0, 0)),
            pl.BlockSpec((1, 1), lambda i: (0, 0)),
        ],
        out_specs=pl.BlockSpec((_BLK, D), lambda i: (i, 0)),
        out_shape=jax.ShapeDtypeStruct((N, D), jnp.float32),
    )(hs[0], hs[1], hs[2], att_w, att_b)


def _pad_edges(ei, ea):
    src = ei[0].astype(jnp.int32)
    dst = ei[1].astype(jnp.int32)
    pad = E_PAD - E
    srcp = jnp.concatenate([src, jnp.zeros((pad,), jnp.int32)]).reshape(32, GROUPS, GSZ)
    dstp = jnp.concatenate([dst, jnp.zeros((pad,), jnp.int32)]).reshape(32, GROUPS, GSZ)
    wp = jnp.concatenate([ea, jnp.zeros((pad,), jnp.float32)]).reshape(32, GROUPS, GSZ)
    return srcp, dstp, wp


def kernel(init, edge_index_cc, edge_attr_cc, edge_index_cac, edge_attr_cac,
           edge_index_csc, edge_attr_csc, params):
    p = params
    x0 = _projection(init, p)
    hs = []
    for g, ei, ea in (('cc', edge_index_cc, edge_attr_cc),
                      ('cac', edge_index_cac, edge_attr_cac),
                      ('csc', edge_index_csc, edge_attr_csc)):
        srcp, dstp, wp = _pad_edges(ei, ea)
        degp = _deg_call(dstp, wp)
        dinv = lax.rsqrt(1.0 + degp[0, :N] + degp[1, :N])[:, None]
        y = _scale_matmul(x0, dinv, p[f'{g}_W0'].T)
        for l in range(3):
            parts = _agg_call(y, srcp, dstp, wp)
            b = p[f'{g}_b{l}'][None, :]
            if l < 2:
                y = _layer(parts, y, dinv, b, p[f'{g}_W{l + 1}'].T)
            else:
                hs.append(_layer_last(parts, y, dinv, b))
    return _attention(hs, p['att_W'], p['att_b'][None, :][:, :1])
